# Initial kernel scaffold; baseline (speedup 1.0000x reference)
import jax
import jax.numpy as jnp
from jax.experimental import pallas as pl


def kernel(x, edge_index, edge_attr, params):
    raise NotImplementedError("write your pallas kernel here")


# speedup vs baseline: 2.0574x; 2.0574x over previous
import functools

import jax
import jax.numpy as jnp
from jax import lax
from jax.experimental import pallas as pl
from jax.experimental.pallas import tpu as pltpu
from jax.experimental.pallas import tpu_sc as plsc

N = 10000
E = 320000
D_IN = 128
D_EDGE = 16
H = 128

NC = 2
NS = 16
NW = NC * NS
E_TILE = E // NW
CH = 80
NCHUNK = E_TILE // CH
ZR = 200
NZCHUNK = N // ZR


def _sc_agg_body(h_hbm, e_hbm, src_hbm, dst_hbm, out_hbm,
                 idx_s, idx_d, rows, ebuf, zbuf, acc, sem_a, sem_b):
    core = lax.axis_index("c")
    sub = lax.axis_index("s")
    wid = core * NS + sub

    def zbody(i, _):
        zbuf[i // 8, pl.ds((i % 8) * 16, 16)] = jnp.zeros((16,), jnp.float32)
        return 0
    lax.fori_loop(0, ZR * 8, zbody, 0, unroll=8)
    for k in range((NZCHUNK + NS - 1) // NS):
        cid = sub + k * NS

        @pl.when(cid < NZCHUNK)
        def _():
            pltpu.sync_copy(zbuf, acc.at[pl.ds(cid * ZR, ZR)])
    plsc.subcore_barrier()

    ebase = wid * E_TILE

    def chunk_body(c, _):
        b = ebase + c * CH
        pltpu.sync_copy(src_hbm.at[pl.ds(b, CH)], idx_s)
        cp_r = pltpu.async_copy(h_hbm.at[idx_s], rows, sem_a)
        cp_e = pltpu.async_copy(e_hbm.at[pl.ds(b, CH)], ebuf, sem_b)
        pltpu.sync_copy(dst_hbm.at[pl.ds(b, CH)], idx_d)
        cp_r.wait()
        cp_e.wait()

        def row_body(r, _):
            for j in range(8):
                sl = pl.ds(j * 16, 16)
                rows[r, sl] = jnp.maximum(rows[r, sl] + ebuf[r, sl], 0.0)
            return 0
        lax.fori_loop(0, CH, row_body, 0, unroll=4)
        pltpu.sync_copy(rows, acc.at[idx_d], add=True)
        return 0

    lax.fori_loop(0, NCHUNK, chunk_body, 0)
    plsc.subcore_barrier()

    for k in range((NZCHUNK + NS - 1) // NS):
        cid = sub + k * NS

        @pl.when(cid < NZCHUNK)
        def _():
            pltpu.sync_copy(acc.at[pl.ds(cid * ZR, ZR)], zbuf)
            pltpu.sync_copy(zbuf, out_hbm.at[pl.ds(core * N + cid * ZR, ZR)])


@functools.cache
def _sc_agg_kernel():
    return pl.kernel(
        _sc_agg_body,
        out_type=jax.ShapeDtypeStruct((NC * N, H), jnp.float32),
        mesh=plsc.VectorSubcoreMesh(core_axis_name="c", subcore_axis_name="s"),
        scratch_types=[
            pltpu.VMEM((CH,), jnp.int32),
            pltpu.VMEM((CH,), jnp.int32),
            pltpu.VMEM((CH, H), jnp.float32),
            pltpu.VMEM((CH, H), jnp.float32),
            pltpu.VMEM((ZR, H), jnp.float32),
            pltpu.VMEM_SHARED((N, H), jnp.float32),
            pltpu.SemaphoreType.DMA,
            pltpu.SemaphoreType.DMA,
        ],
    )


def _sc_agg(h, e, src, dst):
    return _sc_agg_kernel()(h, e, src, dst)


def _dot(a, b):
    return jax.lax.dot_general(a, b, (((1,), (0,)), ((), ())),
                               preferred_element_type=jnp.float32)


def _edge_mlp_body(a_ref, w1_ref, b1_ref, w2_ref, b2_ref, o_ref):
    m = jnp.maximum(_dot(a_ref[...], w1_ref[...]) + b1_ref[...], 0.0)
    o_ref[...] = _dot(m, w2_ref[...]) + b2_ref[...]


def _edge_mlp(edge_attr, p):
    rb = 2000
    return pl.pallas_call(
        _edge_mlp_body,
        grid=(E // rb,),
        in_specs=[
            pl.BlockSpec((rb, D_EDGE), lambda i: (i, 0)),
            pl.BlockSpec((D_EDGE, H), lambda i: (0, 0)),
            pl.BlockSpec((1, H), lambda i: (0, 0)),
            pl.BlockSpec((H, H), lambda i: (0, 0)),
            pl.BlockSpec((1, H), lambda i: (0, 0)),
        ],
        out_specs=pl.BlockSpec((rb, H), lambda i: (i, 0)),
        out_shape=jax.ShapeDtypeStruct((E, H), jnp.float32),
    )(edge_attr, p["W1"], p["b1"].reshape(1, H), p["W2"], p["b2"].reshape(1, H))


def _proj_body(x_ref, w_ref, b_ref, o_ref):
    o_ref[...] = _dot(x_ref[...], w_ref[...]) + b_ref[...]


def _proj(x, W, b):
    rb = 2000
    return pl.pallas_call(
        _proj_body,
        grid=(N // rb,),
        in_specs=[
            pl.BlockSpec((rb, D_IN), lambda i: (i, 0)),
            pl.BlockSpec((D_IN, H), lambda i: (0, 0)),
            pl.BlockSpec((1, H), lambda i: (0, 0)),
        ],
        out_specs=pl.BlockSpec((rb, H), lambda i: (i, 0)),
        out_shape=jax.ShapeDtypeStruct((N, H), jnp.float32),
    )(x, W, b.reshape(1, H))


def _stage_a_body(h_ref, agg_ref, eps_ref, w1_ref, b1_ref, w2_ref, b2_ref,
                  t_ref, s_ref, q_ref):
    k = pl.program_id(0)
    u = h_ref[...] * eps_ref[...] + agg_ref[0] + agg_ref[1]
    m = jnp.maximum(_dot(u, w1_ref[...]) + b1_ref[...], 0.0)
    t = _dot(m, w2_ref[...]) + b2_ref[...]
    t_ref[...] = t
    s = jnp.broadcast_to(jnp.sum(t, axis=0, keepdims=True), (8, H))
    q = jnp.broadcast_to(jnp.sum(t * t, axis=0, keepdims=True), (8, H))

    @pl.when(k == 0)
    def _():
        s_ref[...] = s
        q_ref[...] = q

    @pl.when(k > 0)
    def _():
        s_ref[...] += s
        q_ref[...] += q


def _stage_a(h, agg2, epsp1, p):
    rb = 2000
    return pl.pallas_call(
        _stage_a_body,
        grid=(N // rb,),
        in_specs=[
            pl.BlockSpec((rb, H), lambda i: (i, 0)),
            pl.BlockSpec((2, rb, H), lambda i: (0, i, 0)),
            pl.BlockSpec((1, H), lambda i: (0, 0)),
            pl.BlockSpec((H, H), lambda i: (0, 0)),
            pl.BlockSpec((1, H), lambda i: (0, 0)),
            pl.BlockSpec((H, H), lambda i: (0, 0)),
            pl.BlockSpec((1, H), lambda i: (0, 0)),
        ],
        out_specs=[
            pl.BlockSpec((rb, H), lambda i: (i, 0)),
            pl.BlockSpec((8, H), lambda i: (0, 0)),
            pl.BlockSpec((8, H), lambda i: (0, 0)),
        ],
        out_shape=[
            jax.ShapeDtypeStruct((N, H), jnp.float32),
            jax.ShapeDtypeStruct((8, H), jnp.float32),
            jax.ShapeDtypeStruct((8, H), jnp.float32),
        ],
    )(h, agg2, epsp1, p["W1"], p["b1"].reshape(1, H), p["W2"],
      p["b2"].reshape(1, H))


def _stage_b_body(t_ref, s_ref, q_ref, g_ref, bt_ref, hres_ref, o_ref):
    mean = s_ref[0:1, :] * (1.0 / N)
    var = q_ref[0:1, :] * (1.0 / N) - mean * mean
    scale = jax.lax.rsqrt(var + 1e-5) * g_ref[...]
    o_ref[...] = (jnp.maximum((t_ref[...] - mean) * scale + bt_ref[...], 0.0)
                  + hres_ref[...])


def _stage_b(t, s, q, gamma, beta, h_res):
    rb = 2000
    return pl.pallas_call(
        _stage_b_body,
        grid=(N // rb,),
        in_specs=[
            pl.BlockSpec((rb, H), lambda i: (i, 0)),
            pl.BlockSpec((8, H), lambda i: (0, 0)),
            pl.BlockSpec((8, H), lambda i: (0, 0)),
            pl.BlockSpec((1, H), lambda i: (0, 0)),
            pl.BlockSpec((1, H), lambda i: (0, 0)),
            pl.BlockSpec((rb, H), lambda i: (i, 0)),
        ],
        out_specs=pl.BlockSpec((rb, H), lambda i: (i, 0)),
        out_shape=jax.ShapeDtypeStruct((N, H), jnp.float32),
    )(t, s, q, gamma.reshape(1, H), beta.reshape(1, H), h_res)


def kernel(x, edge_index, edge_attr, params):
    src = edge_index[0].astype(jnp.int32)
    dst = edge_index[1].astype(jnp.int32)

    h = _proj(x, params["proj_W"], params["proj_b"])
    e = _edge_mlp(edge_attr, params["edge_mlp"])

    for l in range(3):
        p = params["layers"][l]
        agg2 = _sc_agg(h, e, src, dst).reshape(2, N, H)
        epsp1 = jnp.broadcast_to((1.0 + p["eps"]).reshape(1, 1), (1, H))
        t, s, q = _stage_a(h, agg2, epsp1, p["mlp"])
        h = _stage_b(t, s, q, p["gamma"], p["beta"], h)
    return h


# speedup vs baseline: 2.3864x; 1.1599x over previous
import functools

import jax
import jax.numpy as jnp
from jax import lax
from jax.experimental import pallas as pl
from jax.experimental.pallas import tpu as pltpu
from jax.experimental.pallas import tpu_sc as plsc

N = 10000
E = 320000
D_IN = 128
D_EDGE = 16
H = 128

NC = 2
NS = 16
NW = NC * NS
E_TILE = E // NW
CH = 80
NCHUNK = E_TILE // CH
ZR = 80
NZCHUNK = N // ZR


def _sc_agg_body(h_hbm, e_hbm, src_hbm, dst_hbm, out_hbm,
                 idx_s0, idx_s1, idx_d0, idx_d1,
                 rows0, rows1, ebuf0, ebuf1, acc,
                 sem_g0, sem_g1, sem_e0, sem_e1, sem_w0, sem_w1):
    core = lax.axis_index("c")
    sub = lax.axis_index("s")
    wid = core * NS + sub

    def zbody(i, _):
        rows0[i // 8, pl.ds((i % 8) * 16, 16)] = jnp.zeros((16,), jnp.float32)
        return 0
    lax.fori_loop(0, ZR * 8, zbody, 0, unroll=8)
    for k in range((NZCHUNK + NS - 1) // NS):
        cid = sub + k * NS

        @pl.when(cid < NZCHUNK)
        def _():
            pltpu.sync_copy(rows0, acc.at[pl.ds(cid * ZR, ZR)])
    plsc.subcore_barrier()

    ebase = wid * E_TILE
    bufs = ((rows0, ebuf0, sem_g0, sem_e0, sem_w0, idx_s0, idx_d0),
            (rows1, ebuf1, sem_g1, sem_e1, sem_w1, idx_s1, idx_d1))

    def in_descs(c, b):
        rows, ebuf, sg, se = bufs[b][:4]
        isb = bufs[b][5]
        return (pltpu.make_async_copy(h_hbm.at[isb], rows, sg),
                pltpu.make_async_copy(e_hbm.at[pl.ds(ebase + c * CH, CH)],
                                      ebuf, se))

    def issue(c, b):
        isb, idb = bufs[b][5], bufs[b][6]
        b0 = ebase + c * CH
        pltpu.sync_copy(src_hbm.at[pl.ds(b0, CH)], isb)
        pltpu.sync_copy(dst_hbm.at[pl.ds(b0, CH)], idb)
        g, ec = in_descs(c, b)
        g.start()
        ec.start()

    def wait_in(c, b):
        g, ec = in_descs(c, b)
        g.wait()
        ec.wait()

    def compute(b):
        rows, ebuf = bufs[b][0], bufs[b][1]

        def row_body(r, _):
            for j in range(8):
                sl = pl.ds(j * 16, 16)
                rows[r, sl] = jnp.maximum(rows[r, sl] + ebuf[r, sl], 0.0)
            return 0
        lax.fori_loop(0, CH, row_body, 0, unroll=4)

    def scatter_start(c, b):
        rows, sw, idb = bufs[b][0], bufs[b][4], bufs[b][6]
        pltpu.async_copy(rows, acc.at[idb], sw, add=True)

    def scatter_wait(b):
        rows, sw, idb = bufs[b][0], bufs[b][4], bufs[b][6]
        pltpu.make_async_copy(rows, acc.at[idb], sw).wait()

    issue(0, 0)
    issue(1, 1)

    def pair(i, _):
        c0 = 2 * i
        wait_in(c0, 0)
        compute(0)
        scatter_start(c0, 0)

        @pl.when(c0 + 2 < NCHUNK)
        def _():
            scatter_wait(0)
            issue(c0 + 2, 0)

        wait_in(c0 + 1, 1)
        compute(1)
        scatter_start(c0 + 1, 1)

        @pl.when(c0 + 3 < NCHUNK)
        def _():
            scatter_wait(1)
            issue(c0 + 3, 1)
        return 0

    lax.fori_loop(0, NCHUNK // 2, pair, 0)
    wait_in(NCHUNK - 1, 0)
    compute(0)
    scatter_start(NCHUNK - 1, 0)
    scatter_wait(0)
    scatter_wait(1)
    plsc.subcore_barrier()

    for k in range((NZCHUNK + NS - 1) // NS):
        cid = sub + k * NS

        @pl.when(cid < NZCHUNK)
        def _():
            pltpu.sync_copy(acc.at[pl.ds(cid * ZR, ZR)], rows0)
            pltpu.sync_copy(rows0, out_hbm.at[pl.ds(core * N + cid * ZR, ZR)])


@functools.cache
def _sc_agg_kernel():
    return pl.kernel(
        _sc_agg_body,
        out_type=jax.ShapeDtypeStruct((NC * N, H), jnp.float32),
        mesh=plsc.VectorSubcoreMesh(core_axis_name="c", subcore_axis_name="s"),
        scratch_types=[
            pltpu.VMEM((CH,), jnp.int32),
            pltpu.VMEM((CH,), jnp.int32),
            pltpu.VMEM((CH,), jnp.int32),
            pltpu.VMEM((CH,), jnp.int32),
            pltpu.VMEM((CH, H), jnp.float32),
            pltpu.VMEM((CH, H), jnp.float32),
            pltpu.VMEM((CH, H), jnp.float32),
            pltpu.VMEM((CH, H), jnp.float32),
            pltpu.VMEM_SHARED((N, H), jnp.float32),
            pltpu.SemaphoreType.DMA,
            pltpu.SemaphoreType.DMA,
            pltpu.SemaphoreType.DMA,
            pltpu.SemaphoreType.DMA,
            pltpu.SemaphoreType.DMA,
            pltpu.SemaphoreType.DMA,
        ],
    )


def _sc_agg(h, e, src, dst):
    return _sc_agg_kernel()(h, e, src, dst)


def _dot(a, b):
    return jax.lax.dot_general(a, b, (((1,), (0,)), ((), ())),
                               preferred_element_type=jnp.float32)


def _edge_mlp_body(a_ref, w1_ref, b1_ref, w2_ref, b2_ref, o_ref):
    m = jnp.maximum(_dot(a_ref[...], w1_ref[...]) + b1_ref[...], 0.0)
    o_ref[...] = _dot(m, w2_ref[...]) + b2_ref[...]


def _edge_mlp(edge_attr, p):
    rb = 2000
    return pl.pallas_call(
        _edge_mlp_body,
        grid=(E // rb,),
        in_specs=[
            pl.BlockSpec((rb, D_EDGE), lambda i: (i, 0)),
            pl.BlockSpec((D_EDGE, H), lambda i: (0, 0)),
            pl.BlockSpec((1, H), lambda i: (0, 0)),
            pl.BlockSpec((H, H), lambda i: (0, 0)),
            pl.BlockSpec((1, H), lambda i: (0, 0)),
        ],
        out_specs=pl.BlockSpec((rb, H), lambda i: (i, 0)),
        out_shape=jax.ShapeDtypeStruct((E, H), jnp.float32),
    )(edge_attr, p["W1"], p["b1"].reshape(1, H), p["W2"], p["b2"].reshape(1, H))


def _proj_body(x_ref, w_ref, b_ref, o_ref):
    o_ref[...] = _dot(x_ref[...], w_ref[...]) + b_ref[...]


def _proj(x, W, b):
    rb = 2000
    return pl.pallas_call(
        _proj_body,
        grid=(N // rb,),
        in_specs=[
            pl.BlockSpec((rb, D_IN), lambda i: (i, 0)),
            pl.BlockSpec((D_IN, H), lambda i: (0, 0)),
            pl.BlockSpec((1, H), lambda i: (0, 0)),
        ],
        out_specs=pl.BlockSpec((rb, H), lambda i: (i, 0)),
        out_shape=jax.ShapeDtypeStruct((N, H), jnp.float32),
    )(x, W, b.reshape(1, H))


def _stage_a_body(h_ref, agg_ref, eps_ref, w1_ref, b1_ref, w2_ref, b2_ref,
                  t_ref, s_ref, q_ref):
    k = pl.program_id(0)
    u = h_ref[...] * eps_ref[...] + agg_ref[0] + agg_ref[1]
    m = jnp.maximum(_dot(u, w1_ref[...]) + b1_ref[...], 0.0)
    t = _dot(m, w2_ref[...]) + b2_ref[...]
    t_ref[...] = t
    s = jnp.broadcast_to(jnp.sum(t, axis=0, keepdims=True), (8, H))
    q = jnp.broadcast_to(jnp.sum(t * t, axis=0, keepdims=True), (8, H))

    @pl.when(k == 0)
    def _():
        s_ref[...] = s
        q_ref[...] = q

    @pl.when(k > 0)
    def _():
        s_ref[...] += s
        q_ref[...] += q


def _stage_a(h, agg2, epsp1, p):
    rb = 2000
    return pl.pallas_call(
        _stage_a_body,
        grid=(N // rb,),
        in_specs=[
            pl.BlockSpec((rb, H), lambda i: (i, 0)),
            pl.BlockSpec((2, rb, H), lambda i: (0, i, 0)),
            pl.BlockSpec((1, H), lambda i: (0, 0)),
            pl.BlockSpec((H, H), lambda i: (0, 0)),
            pl.BlockSpec((1, H), lambda i: (0, 0)),
            pl.BlockSpec((H, H), lambda i: (0, 0)),
            pl.BlockSpec((1, H), lambda i: (0, 0)),
        ],
        out_specs=[
            pl.BlockSpec((rb, H), lambda i: (i, 0)),
            pl.BlockSpec((8, H), lambda i: (0, 0)),
            pl.BlockSpec((8, H), lambda i: (0, 0)),
        ],
        out_shape=[
            jax.ShapeDtypeStruct((N, H), jnp.float32),
            jax.ShapeDtypeStruct((8, H), jnp.float32),
            jax.ShapeDtypeStruct((8, H), jnp.float32),
        ],
    )(h, agg2, epsp1, p["W1"], p["b1"].reshape(1, H), p["W2"],
      p["b2"].reshape(1, H))


def _stage_b_body(t_ref, s_ref, q_ref, g_ref, bt_ref, hres_ref, o_ref):
    mean = s_ref[0:1, :] * (1.0 / N)
    var = q_ref[0:1, :] * (1.0 / N) - mean * mean
    scale = jax.lax.rsqrt(var + 1e-5) * g_ref[...]
    o_ref[...] = (jnp.maximum((t_ref[...] - mean) * scale + bt_ref[...], 0.0)
                  + hres_ref[...])


def _stage_b(t, s, q, gamma, beta, h_res):
    rb = 2000
    return pl.pallas_call(
        _stage_b_body,
        grid=(N // rb,),
        in_specs=[
            pl.BlockSpec((rb, H), lambda i: (i, 0)),
            pl.BlockSpec((8, H), lambda i: (0, 0)),
            pl.BlockSpec((8, H), lambda i: (0, 0)),
            pl.BlockSpec((1, H), lambda i: (0, 0)),
            pl.BlockSpec((1, H), lambda i: (0, 0)),
            pl.BlockSpec((rb, H), lambda i: (i, 0)),
        ],
        out_specs=pl.BlockSpec((rb, H), lambda i: (i, 0)),
        out_shape=jax.ShapeDtypeStruct((N, H), jnp.float32),
    )(t, s, q, gamma.reshape(1, H), beta.reshape(1, H), h_res)


def kernel(x, edge_index, edge_attr, params):
    src = edge_index[0].astype(jnp.int32)
    dst = edge_index[1].astype(jnp.int32)

    h = _proj(x, params["proj_W"], params["proj_b"])
    e = _edge_mlp(edge_attr, params["edge_mlp"])

    for l in range(3):
        p = params["layers"][l]
        agg2 = _sc_agg(h, e, src, dst).reshape(2, N, H)
        epsp1 = jnp.broadcast_to((1.0 + p["eps"]).reshape(1, 1), (1, H))
        t, s, q = _stage_a(h, agg2, epsp1, p["mlp"])
        h = _stage_b(t, s, q, p["gamma"], p["beta"], h)
    return h


# speedup vs baseline: 4.2828x; 1.7947x over previous
import functools

import jax
import jax.numpy as jnp
from jax import lax
from jax.experimental import pallas as pl
from jax.experimental.pallas import tpu as pltpu
from jax.experimental.pallas import tpu_sc as plsc

N = 10000
E = 320000
D_IN = 128
D_EDGE = 16
H = 128

NC = 2
NS = 16
NW = NC * NS
E_TILE = E // NW
CH = 80
NCHUNK = E_TILE // CH
ZR = 80
NZCHUNK = N // ZR


def _sc_agg_body(h_hbm, e_hbm, src_hbm, dst_hbm, out_hbm,
                 idx_s0, idx_s1, idx_d0, idx_d1,
                 rows0, rows1, ebuf0, ebuf1, acc,
                 sem_g0, sem_g1, sem_e0, sem_e1, sem_w0, sem_w1):
    core = lax.axis_index("c")
    sub = lax.axis_index("s")
    wid = core * NS + sub

    def zbody(i, _):
        rows0[i // 8, pl.ds((i % 8) * 16, 16)] = jnp.zeros((16,), jnp.float32)
        return 0
    lax.fori_loop(0, ZR * 8, zbody, 0, unroll=8)
    for k in range((NZCHUNK + NS - 1) // NS):
        cid = sub + k * NS

        @pl.when(cid < NZCHUNK)
        def _():
            pltpu.sync_copy(rows0, acc.at[pl.ds(cid * ZR, ZR)])
    plsc.subcore_barrier()

    ebase = wid * E_TILE
    bufs = ((rows0, ebuf0, sem_g0, sem_e0, sem_w0, idx_s0, idx_d0),
            (rows1, ebuf1, sem_g1, sem_e1, sem_w1, idx_s1, idx_d1))

    def in_descs(c, b):
        rows, ebuf, sg, se = bufs[b][:4]
        isb = bufs[b][5]
        return (pltpu.make_async_copy(h_hbm.at[isb], rows, sg),
                pltpu.make_async_copy(e_hbm.at[pl.ds(ebase + c * CH, CH)],
                                      ebuf, se))

    def issue(c, b):
        isb, idb = bufs[b][5], bufs[b][6]
        b0 = ebase + c * CH
        pltpu.sync_copy(src_hbm.at[pl.ds(b0, CH)], isb)
        pltpu.sync_copy(dst_hbm.at[pl.ds(b0, CH)], idb)
        g, ec = in_descs(c, b)
        g.start()
        ec.start()

    def wait_in(c, b):
        g, ec = in_descs(c, b)
        g.wait()
        ec.wait()

    def compute(b):
        rows, ebuf = bufs[b][0], bufs[b][1]

        @plsc.parallel_loop(0, CH, 1, unroll=4)
        def row_body(r):
            for j in range(8):
                sl = pl.ds(j * 16, 16)
                rows[r, sl] = jnp.maximum(rows[r, sl] + ebuf[r, sl], 0.0)

    def scatter_start(c, b):
        rows, sw, idb = bufs[b][0], bufs[b][4], bufs[b][6]
        pltpu.async_copy(rows, acc.at[idb], sw, add=True)

    def scatter_wait(b):
        rows, sw, idb = bufs[b][0], bufs[b][4], bufs[b][6]
        pltpu.make_async_copy(rows, acc.at[idb], sw).wait()

    issue(0, 0)
    issue(1, 1)

    def pair(i, _):
        c0 = 2 * i
        wait_in(c0, 0)
        compute(0)
        scatter_start(c0, 0)

        @pl.when(c0 + 2 < NCHUNK)
        def _():
            scatter_wait(0)
            issue(c0 + 2, 0)

        wait_in(c0 + 1, 1)
        compute(1)
        scatter_start(c0 + 1, 1)

        @pl.when(c0 + 3 < NCHUNK)
        def _():
            scatter_wait(1)
            issue(c0 + 3, 1)
        return 0

    lax.fori_loop(0, NCHUNK // 2, pair, 0)
    wait_in(NCHUNK - 1, 0)
    compute(0)
    scatter_start(NCHUNK - 1, 0)
    scatter_wait(0)
    scatter_wait(1)
    plsc.subcore_barrier()

    for k in range((NZCHUNK + NS - 1) // NS):
        cid = sub + k * NS

        @pl.when(cid < NZCHUNK)
        def _():
            pltpu.sync_copy(acc.at[pl.ds(cid * ZR, ZR)], rows0)
            pltpu.sync_copy(rows0, out_hbm.at[pl.ds(core * N + cid * ZR, ZR)])


@functools.cache
def _sc_agg_kernel():
    return pl.kernel(
        _sc_agg_body,
        out_type=jax.ShapeDtypeStruct((NC * N, H), jnp.float32),
        mesh=plsc.VectorSubcoreMesh(core_axis_name="c", subcore_axis_name="s"),
        scratch_types=[
            pltpu.VMEM((CH,), jnp.int32),
            pltpu.VMEM((CH,), jnp.int32),
            pltpu.VMEM((CH,), jnp.int32),
            pltpu.VMEM((CH,), jnp.int32),
            pltpu.VMEM((CH, H), jnp.float32),
            pltpu.VMEM((CH, H), jnp.float32),
            pltpu.VMEM((CH, H), jnp.float32),
            pltpu.VMEM((CH, H), jnp.float32),
            pltpu.VMEM_SHARED((N, H), jnp.float32),
            pltpu.SemaphoreType.DMA,
            pltpu.SemaphoreType.DMA,
            pltpu.SemaphoreType.DMA,
            pltpu.SemaphoreType.DMA,
            pltpu.SemaphoreType.DMA,
            pltpu.SemaphoreType.DMA,
        ],
    )


def _sc_agg(h, e, src, dst):
    return _sc_agg_kernel()(h, e, src, dst)


def _dot(a, b):
    return jax.lax.dot_general(a, b, (((1,), (0,)), ((), ())),
                               preferred_element_type=jnp.float32)


def _edge_mlp_body(a_ref, w1_ref, b1_ref, w2_ref, b2_ref, o_ref):
    m = jnp.maximum(_dot(a_ref[...], w1_ref[...]) + b1_ref[...], 0.0)
    o_ref[...] = _dot(m, w2_ref[...]) + b2_ref[...]


def _edge_mlp(edge_attr, p):
    rb = 2000
    return pl.pallas_call(
        _edge_mlp_body,
        grid=(E // rb,),
        in_specs=[
            pl.BlockSpec((rb, D_EDGE), lambda i: (i, 0)),
            pl.BlockSpec((D_EDGE, H), lambda i: (0, 0)),
            pl.BlockSpec((1, H), lambda i: (0, 0)),
            pl.BlockSpec((H, H), lambda i: (0, 0)),
            pl.BlockSpec((1, H), lambda i: (0, 0)),
        ],
        out_specs=pl.BlockSpec((rb, H), lambda i: (i, 0)),
        out_shape=jax.ShapeDtypeStruct((E, H), jnp.float32),
    )(edge_attr, p["W1"], p["b1"].reshape(1, H), p["W2"], p["b2"].reshape(1, H))


def _proj_body(x_ref, w_ref, b_ref, o_ref):
    o_ref[...] = _dot(x_ref[...], w_ref[...]) + b_ref[...]


def _proj(x, W, b):
    rb = 2000
    return pl.pallas_call(
        _proj_body,
        grid=(N // rb,),
        in_specs=[
            pl.BlockSpec((rb, D_IN), lambda i: (i, 0)),
            pl.BlockSpec((D_IN, H), lambda i: (0, 0)),
            pl.BlockSpec((1, H), lambda i: (0, 0)),
        ],
        out_specs=pl.BlockSpec((rb, H), lambda i: (i, 0)),
        out_shape=jax.ShapeDtypeStruct((N, H), jnp.float32),
    )(x, W, b.reshape(1, H))


def _stage_a_body(h_ref, agg_ref, eps_ref, w1_ref, b1_ref, w2_ref, b2_ref,
                  t_ref, s_ref, q_ref):
    k = pl.program_id(0)
    u = h_ref[...] * eps_ref[...] + agg_ref[0] + agg_ref[1]
    m = jnp.maximum(_dot(u, w1_ref[...]) + b1_ref[...], 0.0)
    t = _dot(m, w2_ref[...]) + b2_ref[...]
    t_ref[...] = t
    s = jnp.broadcast_to(jnp.sum(t, axis=0, keepdims=True), (8, H))
    q = jnp.broadcast_to(jnp.sum(t * t, axis=0, keepdims=True), (8, H))

    @pl.when(k == 0)
    def _():
        s_ref[...] = s
        q_ref[...] = q

    @pl.when(k > 0)
    def _():
        s_ref[...] += s
        q_ref[...] += q


def _stage_a(h, agg2, epsp1, p):
    rb = 2000
    return pl.pallas_call(
        _stage_a_body,
        grid=(N // rb,),
        in_specs=[
            pl.BlockSpec((rb, H), lambda i: (i, 0)),
            pl.BlockSpec((2, rb, H), lambda i: (0, i, 0)),
            pl.BlockSpec((1, H), lambda i: (0, 0)),
            pl.BlockSpec((H, H), lambda i: (0, 0)),
            pl.BlockSpec((1, H), lambda i: (0, 0)),
            pl.BlockSpec((H, H), lambda i: (0, 0)),
            pl.BlockSpec((1, H), lambda i: (0, 0)),
        ],
        out_specs=[
            pl.BlockSpec((rb, H), lambda i: (i, 0)),
            pl.BlockSpec((8, H), lambda i: (0, 0)),
            pl.BlockSpec((8, H), lambda i: (0, 0)),
        ],
        out_shape=[
            jax.ShapeDtypeStruct((N, H), jnp.float32),
            jax.ShapeDtypeStruct((8, H), jnp.float32),
            jax.ShapeDtypeStruct((8, H), jnp.float32),
        ],
    )(h, agg2, epsp1, p["W1"], p["b1"].reshape(1, H), p["W2"],
      p["b2"].reshape(1, H))


def _stage_b_body(t_ref, s_ref, q_ref, g_ref, bt_ref, hres_ref, o_ref):
    mean = s_ref[0:1, :] * (1.0 / N)
    var = q_ref[0:1, :] * (1.0 / N) - mean * mean
    scale = jax.lax.rsqrt(var + 1e-5) * g_ref[...]
    o_ref[...] = (jnp.maximum((t_ref[...] - mean) * scale + bt_ref[...], 0.0)
                  + hres_ref[...])


def _stage_b(t, s, q, gamma, beta, h_res):
    rb = 2000
    return pl.pallas_call(
        _stage_b_body,
        grid=(N // rb,),
        in_specs=[
            pl.BlockSpec((rb, H), lambda i: (i, 0)),
            pl.BlockSpec((8, H), lambda i: (0, 0)),
            pl.BlockSpec((8, H), lambda i: (0, 0)),
            pl.BlockSpec((1, H), lambda i: (0, 0)),
            pl.BlockSpec((1, H), lambda i: (0, 0)),
            pl.BlockSpec((rb, H), lambda i: (i, 0)),
        ],
        out_specs=pl.BlockSpec((rb, H), lambda i: (i, 0)),
        out_shape=jax.ShapeDtypeStruct((N, H), jnp.float32),
    )(t, s, q, gamma.reshape(1, H), beta.reshape(1, H), h_res)


def kernel(x, edge_index, edge_attr, params):
    src = edge_index[0].astype(jnp.int32)
    dst = edge_index[1].astype(jnp.int32)

    h = _proj(x, params["proj_W"], params["proj_b"])
    e = _edge_mlp(edge_attr, params["edge_mlp"])

    for l in range(3):
        p = params["layers"][l]
        agg2 = _sc_agg(h, e, src, dst).reshape(2, N, H)
        epsp1 = jnp.broadcast_to((1.0 + p["eps"]).reshape(1, 1), (1, H))
        t, s, q = _stage_a(h, agg2, epsp1, p["mlp"])
        h = _stage_b(t, s, q, p["gamma"], p["beta"], h)
    return h


# speedup vs baseline: 5.1642x; 1.2058x over previous
import functools

import jax
import jax.numpy as jnp
from jax import lax
from jax.experimental import pallas as pl
from jax.experimental.pallas import tpu as pltpu
from jax.experimental.pallas import tpu_sc as plsc

N = 10000
E = 320000
D_IN = 128
D_EDGE = 16
H = 128

NC = 2
NS = 16
NW = NC * NS
E_TILE = E // NW
CH = 80
NCHUNK = E_TILE // CH
ZR = 80
NZCHUNK = N // ZR


def _sc_agg_body(h_hbm, e_hbm, src_hbm, dst_hbm, out_hbm,
                 idx_s, idx_d, rows0, rows1, ebuf0, ebuf1, acc,
                 sem_g0, sem_g1, sem_e0, sem_e1, sem_w0, sem_w1,
                 sem_i0, sem_i1, sem_i2, sem_i3):
    core = lax.axis_index("c")
    sub = lax.axis_index("s")
    wid = core * NS + sub

    def zbody(i, _):
        rows0[i // 8, pl.ds((i % 8) * 16, 16)] = jnp.zeros((16,), jnp.float32)
        return 0
    lax.fori_loop(0, ZR * 8, zbody, 0, unroll=8)
    for k in range((NZCHUNK + NS - 1) // NS):
        cid = sub + k * NS

        @pl.when(cid < NZCHUNK)
        def _():
            pltpu.sync_copy(rows0, acc.at[pl.ds(cid * ZR, ZR)])
    plsc.subcore_barrier()

    ebase = wid * E_TILE
    bufs = ((rows0, ebuf0, sem_g0, sem_e0, sem_w0),
            (rows1, ebuf1, sem_g1, sem_e1, sem_w1))
    isems = (sem_i0, sem_i1, sem_i2, sem_i3)

    def idx_descs(c, q):
        b0 = ebase + c * CH
        return (pltpu.make_async_copy(src_hbm.at[pl.ds(b0, CH)],
                                      idx_s.at[q], isems[q]),
                pltpu.make_async_copy(dst_hbm.at[pl.ds(b0, CH)],
                                      idx_d.at[q], isems[q]))

    def idx_start(c, q):
        s, d = idx_descs(c, q)
        s.start()
        d.start()

    def idx_wait(c, q):
        s, d = idx_descs(c, q)
        s.wait()
        d.wait()

    def in_descs(c, b, q):
        rows, ebuf, sg, se = bufs[b][:4]
        return (pltpu.make_async_copy(h_hbm.at[idx_s.at[q]], rows, sg),
                pltpu.make_async_copy(e_hbm.at[pl.ds(ebase + c * CH, CH)],
                                      ebuf, se))

    def issue(c, b, q):
        g, ec = in_descs(c, b, q)
        g.start()
        ec.start()

    def wait_in(c, b, q):
        g, ec = in_descs(c, b, q)
        g.wait()
        ec.wait()

    def compute(b):
        rows, ebuf = bufs[b][0], bufs[b][1]

        @plsc.parallel_loop(0, CH, 1, unroll=8)
        def row_body(r):
            for j in range(8):
                sl = pl.ds(j * 16, 16)
                rows[r, sl] = jnp.maximum(rows[r, sl] + ebuf[r, sl], 0.0)

    def scatter_start(c, b, q):
        rows, sw = bufs[b][0], bufs[b][4]
        pltpu.async_copy(rows, acc.at[idx_d.at[q]], sw, add=True)

    def scatter_wait(b, q):
        rows, sw = bufs[b][0], bufs[b][4]
        pltpu.make_async_copy(rows, acc.at[idx_d.at[q]], sw).wait()

    def step(c, b, q):
        wait_in(c, b, q)
        compute(b)
        scatter_start(c, b, q)

        @pl.when(c + 2 < NCHUNK)
        def _():
            scatter_wait(b, q)
            idx_wait(c + 2, (q + 2) % 4)
            issue(c + 2, b, (q + 2) % 4)

        @pl.when(c + 4 < NCHUNK)
        def _():
            idx_start(c + 4, q)

    for q in range(4):
        idx_start(q, q)
    idx_wait(0, 0)
    issue(0, 0, 0)
    idx_wait(1, 1)
    issue(1, 1, 1)

    def quad(i, _):
        c0 = 4 * i
        step(c0, 0, 0)
        step(c0 + 1, 1, 1)
        step(c0 + 2, 0, 2)
        step(c0 + 3, 1, 3)
        return 0

    assert NCHUNK % 4 == 1
    lax.fori_loop(0, NCHUNK // 4, quad, 0)
    wait_in(NCHUNK - 1, 0, 0)
    compute(0)
    scatter_start(NCHUNK - 1, 0, 0)
    scatter_wait(0, 0)
    scatter_wait(1, 3)
    plsc.subcore_barrier()

    for k in range((NZCHUNK + NS - 1) // NS):
        cid = sub + k * NS

        @pl.when(cid < NZCHUNK)
        def _():
            pltpu.sync_copy(acc.at[pl.ds(cid * ZR, ZR)], rows0)
            pltpu.sync_copy(rows0, out_hbm.at[pl.ds(core * N + cid * ZR, ZR)])


@functools.cache
def _sc_agg_kernel():
    return pl.kernel(
        _sc_agg_body,
        out_type=jax.ShapeDtypeStruct((NC * N, H), jnp.float32),
        mesh=plsc.VectorSubcoreMesh(core_axis_name="c", subcore_axis_name="s"),
        scratch_types=[
            pltpu.VMEM((4, CH), jnp.int32),
            pltpu.VMEM((4, CH), jnp.int32),
            pltpu.VMEM((CH, H), jnp.float32),
            pltpu.VMEM((CH, H), jnp.float32),
            pltpu.VMEM((CH, H), jnp.float32),
            pltpu.VMEM((CH, H), jnp.float32),
            pltpu.VMEM_SHARED((N, H), jnp.float32),
            pltpu.SemaphoreType.DMA,
            pltpu.SemaphoreType.DMA,
            pltpu.SemaphoreType.DMA,
            pltpu.SemaphoreType.DMA,
            pltpu.SemaphoreType.DMA,
            pltpu.SemaphoreType.DMA,
            pltpu.SemaphoreType.DMA,
            pltpu.SemaphoreType.DMA,
            pltpu.SemaphoreType.DMA,
            pltpu.SemaphoreType.DMA,
        ],
    )


def _sc_agg(h, e, src, dst):
    return _sc_agg_kernel()(h, e, src, dst)


def _dot(a, b):
    return jax.lax.dot_general(a, b, (((1,), (0,)), ((), ())),
                               preferred_element_type=jnp.float32)


def _edge_mlp_body(a_ref, w1_ref, b1_ref, w2_ref, b2_ref, o_ref):
    m = jnp.maximum(_dot(a_ref[...], w1_ref[...]) + b1_ref[...], 0.0)
    o_ref[...] = _dot(m, w2_ref[...]) + b2_ref[...]


def _edge_mlp(edge_attr, p):
    rb = 2000
    return pl.pallas_call(
        _edge_mlp_body,
        grid=(E // rb,),
        in_specs=[
            pl.BlockSpec((rb, D_EDGE), lambda i: (i, 0)),
            pl.BlockSpec((D_EDGE, H), lambda i: (0, 0)),
            pl.BlockSpec((1, H), lambda i: (0, 0)),
            pl.BlockSpec((H, H), lambda i: (0, 0)),
            pl.BlockSpec((1, H), lambda i: (0, 0)),
        ],
        out_specs=pl.BlockSpec((rb, H), lambda i: (i, 0)),
        out_shape=jax.ShapeDtypeStruct((E, H), jnp.float32),
    )(edge_attr, p["W1"], p["b1"].reshape(1, H), p["W2"], p["b2"].reshape(1, H))


def _proj_body(x_ref, w_ref, b_ref, o_ref):
    o_ref[...] = _dot(x_ref[...], w_ref[...]) + b_ref[...]


def _proj(x, W, b):
    rb = 2000
    return pl.pallas_call(
        _proj_body,
        grid=(N // rb,),
        in_specs=[
            pl.BlockSpec((rb, D_IN), lambda i: (i, 0)),
            pl.BlockSpec((D_IN, H), lambda i: (0, 0)),
            pl.BlockSpec((1, H), lambda i: (0, 0)),
        ],
        out_specs=pl.BlockSpec((rb, H), lambda i: (i, 0)),
        out_shape=jax.ShapeDtypeStruct((N, H), jnp.float32),
    )(x, W, b.reshape(1, H))


def _stage_a_body(h_ref, agg_ref, eps_ref, w1_ref, b1_ref, w2_ref, b2_ref,
                  t_ref, s_ref, q_ref):
    k = pl.program_id(0)
    u = h_ref[...] * eps_ref[...] + agg_ref[0] + agg_ref[1]
    m = jnp.maximum(_dot(u, w1_ref[...]) + b1_ref[...], 0.0)
    t = _dot(m, w2_ref[...]) + b2_ref[...]
    t_ref[...] = t
    s = jnp.broadcast_to(jnp.sum(t, axis=0, keepdims=True), (8, H))
    q = jnp.broadcast_to(jnp.sum(t * t, axis=0, keepdims=True), (8, H))

    @pl.when(k == 0)
    def _():
        s_ref[...] = s
        q_ref[...] = q

    @pl.when(k > 0)
    def _():
        s_ref[...] += s
        q_ref[...] += q


def _stage_a(h, agg2, epsp1, p):
    rb = 2000
    return pl.pallas_call(
        _stage_a_body,
        grid=(N // rb,),
        in_specs=[
            pl.BlockSpec((rb, H), lambda i: (i, 0)),
            pl.BlockSpec((2, rb, H), lambda i: (0, i, 0)),
            pl.BlockSpec((1, H), lambda i: (0, 0)),
            pl.BlockSpec((H, H), lambda i: (0, 0)),
            pl.BlockSpec((1, H), lambda i: (0, 0)),
            pl.BlockSpec((H, H), lambda i: (0, 0)),
            pl.BlockSpec((1, H), lambda i: (0, 0)),
        ],
        out_specs=[
            pl.BlockSpec((rb, H), lambda i: (i, 0)),
            pl.BlockSpec((8, H), lambda i: (0, 0)),
            pl.BlockSpec((8, H), lambda i: (0, 0)),
        ],
        out_shape=[
            jax.ShapeDtypeStruct((N, H), jnp.float32),
            jax.ShapeDtypeStruct((8, H), jnp.float32),
            jax.ShapeDtypeStruct((8, H), jnp.float32),
        ],
    )(h, agg2, epsp1, p["W1"], p["b1"].reshape(1, H), p["W2"],
      p["b2"].reshape(1, H))


def _stage_b_body(t_ref, s_ref, q_ref, g_ref, bt_ref, hres_ref, o_ref):
    mean = s_ref[0:1, :] * (1.0 / N)
    var = q_ref[0:1, :] * (1.0 / N) - mean * mean
    scale = jax.lax.rsqrt(var + 1e-5) * g_ref[...]
    o_ref[...] = (jnp.maximum((t_ref[...] - mean) * scale + bt_ref[...], 0.0)
                  + hres_ref[...])


def _stage_b(t, s, q, gamma, beta, h_res):
    rb = 2000
    return pl.pallas_call(
        _stage_b_body,
        grid=(N // rb,),
        in_specs=[
            pl.BlockSpec((rb, H), lambda i: (i, 0)),
            pl.BlockSpec((8, H), lambda i: (0, 0)),
            pl.BlockSpec((8, H), lambda i: (0, 0)),
            pl.BlockSpec((1, H), lambda i: (0, 0)),
            pl.BlockSpec((1, H), lambda i: (0, 0)),
            pl.BlockSpec((rb, H), lambda i: (i, 0)),
        ],
        out_specs=pl.BlockSpec((rb, H), lambda i: (i, 0)),
        out_shape=jax.ShapeDtypeStruct((N, H), jnp.float32),
    )(t, s, q, gamma.reshape(1, H), beta.reshape(1, H), h_res)


def kernel(x, edge_index, edge_attr, params):
    src = edge_index[0].astype(jnp.int32)
    dst = edge_index[1].astype(jnp.int32)

    h = _proj(x, params["proj_W"], params["proj_b"])
    e = _edge_mlp(edge_attr, params["edge_mlp"])

    for l in range(3):
        p = params["layers"][l]
        agg2 = _sc_agg(h, e, src, dst).reshape(2, N, H)
        epsp1 = jnp.broadcast_to((1.0 + p["eps"]).reshape(1, 1), (1, H))
        t, s, q = _stage_a(h, agg2, epsp1, p["mlp"])
        h = _stage_b(t, s, q, p["gamma"], p["beta"], h)
    return h


# speedup vs baseline: 5.1808x; 1.0032x over previous
import functools

import jax
import jax.numpy as jnp
import numpy as np
from jax import lax
from jax.experimental import pallas as pl
from jax.experimental.pallas import tpu as pltpu
from jax.experimental.pallas import tpu_sc as plsc

N = 10000
E = 320000
D_IN = 128
D_EDGE = 16
H = 128

NC = 2
NS = 16
NW = NC * NS
E_TILE = E // NW
CH = 80
NCHUNK = E_TILE // CH
ZR = 80
NZCHUNK = N // ZR

_EPERM = np.array([32 * j + (t // 2 if t % 2 == 0 else 16 + t // 2)
                   for j in range(4) for t in range(32)])


def _sc_agg_body(h_hbm, e_hbm, src_hbm, dst_hbm, out_hbm,
                 idx_s, idx_d, rows0, rows1, ebuf0, ebuf1, acc,
                 sem_g0, sem_g1, sem_e0, sem_e1, sem_w0, sem_w1,
                 sem_i0, sem_i1, sem_i2, sem_i3):
    core = lax.axis_index("c")
    sub = lax.axis_index("s")
    wid = core * NS + sub
    ebase = wid * E_TILE
    bufs = ((rows0, ebuf0, sem_g0, sem_e0, sem_w0),
            (rows1, ebuf1, sem_g1, sem_e1, sem_w1))
    isems = (sem_i0, sem_i1, sem_i2, sem_i3)

    def idx_descs(c, q):
        b0 = ebase + c * CH
        return (pltpu.make_async_copy(src_hbm.at[pl.ds(b0, CH)],
                                      idx_s.at[q], isems[q]),
                pltpu.make_async_copy(dst_hbm.at[pl.ds(b0, CH)],
                                      idx_d.at[q], isems[q]))

    def idx_start(c, q):
        s, d = idx_descs(c, q)
        s.start()
        d.start()

    def idx_wait(c, q):
        s, d = idx_descs(c, q)
        s.wait()
        d.wait()

    def in_descs(c, b, q):
        rows, ebuf, sg, se = bufs[b][:4]
        return (pltpu.make_async_copy(h_hbm.at[idx_s.at[q]], rows, sg),
                pltpu.make_async_copy(e_hbm.at[pl.ds(ebase + c * CH, CH)],
                                      ebuf, se))

    def issue(c, b, q):
        g, ec = in_descs(c, b, q)
        g.start()
        ec.start()

    def wait_in(c, b, q):
        g, ec = in_descs(c, b, q)
        g.wait()
        ec.wait()

    def compute(b):
        rows, ebuf = bufs[b][0], bufs[b][1]

        @plsc.parallel_loop(0, CH, 1, unroll=8)
        def row_body(r):
            for j in range(8):
                sl = pl.ds(j * 16, 16)
                rows[r, sl] = jnp.maximum(rows[r, sl] + ebuf[r, sl], 0.0)

    def scatter_start(c, b, q):
        rows, sw = bufs[b][0], bufs[b][4]
        pltpu.async_copy(rows, acc.at[idx_d.at[q]], sw, add=True)

    def scatter_wait(b, q):
        rows, sw = bufs[b][0], bufs[b][4]
        pltpu.make_async_copy(rows, acc.at[idx_d.at[q]], sw).wait()

    def step(c, b, q):
        wait_in(c, b, q)
        compute(b)
        scatter_start(c, b, q)

        @pl.when(c + 2 < NCHUNK)
        def _():
            scatter_wait(b, q)
            idx_wait(c + 2, (q + 2) % 4)
            issue(c + 2, b, (q + 2) % 4)

        @pl.when(c + 4 < NCHUNK)
        def _():
            idx_start(c + 4, q)

    for q in range(4):
        idx_start(q, q)

    def zbody(i, _):
        rows0[i // 8, pl.ds((i % 8) * 16, 16)] = jnp.zeros((16,), jnp.float32)
        return 0
    lax.fori_loop(0, ZR * 8, zbody, 0, unroll=8)
    for k in range((NZCHUNK + NS - 1) // NS):
        cid = sub + k * NS

        @pl.when(cid < NZCHUNK)
        def _():
            pltpu.sync_copy(rows0, acc.at[pl.ds(cid * ZR, ZR)])
    plsc.subcore_barrier()

    idx_wait(0, 0)
    issue(0, 0, 0)
    idx_wait(1, 1)
    issue(1, 1, 1)

    def quad(i, _):
        c0 = 4 * i
        step(c0, 0, 0)
        step(c0 + 1, 1, 1)
        step(c0 + 2, 0, 2)
        step(c0 + 3, 1, 3)
        return 0

    assert NCHUNK % 4 == 1
    lax.fori_loop(0, NCHUNK // 4, quad, 0)
    wait_in(NCHUNK - 1, 0, 0)
    compute(0)
    scatter_start(NCHUNK - 1, 0, 0)
    scatter_wait(0, 0)
    scatter_wait(1, 3)
    plsc.subcore_barrier()

    for k in range((NZCHUNK + NS - 1) // NS):
        cid = sub + k * NS

        @pl.when(cid < NZCHUNK)
        def _():
            pltpu.sync_copy(acc.at[pl.ds(cid * ZR, ZR)], rows0)
            pltpu.sync_copy(rows0, out_hbm.at[pl.ds(core * N + cid * ZR, ZR)])


@functools.cache
def _sc_agg_kernel():
    return pl.kernel(
        _sc_agg_body,
        out_type=jax.ShapeDtypeStruct((NC * N, H), jnp.float32),
        mesh=plsc.VectorSubcoreMesh(core_axis_name="c", subcore_axis_name="s"),
        scratch_types=[
            pltpu.VMEM((4, CH), jnp.int32),
            pltpu.VMEM((4, CH), jnp.int32),
            pltpu.VMEM((CH, H), jnp.float32),
            pltpu.VMEM((CH, H), jnp.float32),
            pltpu.VMEM((CH, H), jnp.float32),
            pltpu.VMEM((CH, H), jnp.float32),
            pltpu.VMEM_SHARED((N, H), jnp.float32),
            pltpu.SemaphoreType.DMA,
            pltpu.SemaphoreType.DMA,
            pltpu.SemaphoreType.DMA,
            pltpu.SemaphoreType.DMA,
            pltpu.SemaphoreType.DMA,
            pltpu.SemaphoreType.DMA,
            pltpu.SemaphoreType.DMA,
            pltpu.SemaphoreType.DMA,
            pltpu.SemaphoreType.DMA,
            pltpu.SemaphoreType.DMA,
        ],
    )


def _sc_agg(h, e, src, dst):
    return _sc_agg_kernel()(h, e, src, dst)


def _dot(a, b):
    return jax.lax.dot_general(a, b, (((1,), (0,)), ((), ())),
                               preferred_element_type=jnp.float32)


def _edge_mlp_body(a_ref, w1_ref, b1_ref, w2_ref, b2_ref, o_ref):
    m = jnp.maximum(_dot(a_ref[...], w1_ref[...]) + b1_ref[...], 0.0)
    o_ref[...] = _dot(m, w2_ref[...]) + b2_ref[...]


def _edge_mlp(edge_attr, p):
    rb = 2000
    return pl.pallas_call(
        _edge_mlp_body,
        grid=(E // rb,),
        in_specs=[
            pl.BlockSpec((rb, D_EDGE), lambda i: (i, 0)),
            pl.BlockSpec((D_EDGE, H), lambda i: (0, 0)),
            pl.BlockSpec((1, H), lambda i: (0, 0)),
            pl.BlockSpec((H, H), lambda i: (0, 0)),
            pl.BlockSpec((1, H), lambda i: (0, 0)),
        ],
        out_specs=pl.BlockSpec((rb, H), lambda i: (i, 0)),
        out_shape=jax.ShapeDtypeStruct((E, H), jnp.float32),
    )(edge_attr, p["W1"], p["b1"].reshape(1, H), p["W2"], p["b2"].reshape(1, H))


def _proj_body(x_ref, w_ref, b_ref, o_ref):
    o_ref[...] = _dot(x_ref[...], w_ref[...]) + b_ref[...]


def _proj(x, W, b):
    rb = 2000
    return pl.pallas_call(
        _proj_body,
        grid=(N // rb,),
        in_specs=[
            pl.BlockSpec((rb, D_IN), lambda i: (i, 0)),
            pl.BlockSpec((D_IN, H), lambda i: (0, 0)),
            pl.BlockSpec((1, H), lambda i: (0, 0)),
        ],
        out_specs=pl.BlockSpec((rb, H), lambda i: (i, 0)),
        out_shape=jax.ShapeDtypeStruct((N, H), jnp.float32),
    )(x, W, b.reshape(1, H))


def _stage_a_body(h_ref, agg_ref, eps_ref, w1_ref, b1_ref, w2_ref, b2_ref,
                  t_ref, s_ref, q_ref):
    k = pl.program_id(0)
    u = h_ref[...] * eps_ref[...] + agg_ref[0] + agg_ref[1]
    m = jnp.maximum(_dot(u, w1_ref[...]) + b1_ref[...], 0.0)
    t = _dot(m, w2_ref[...]) + b2_ref[...]
    t_ref[...] = t
    s = jnp.broadcast_to(jnp.sum(t, axis=0, keepdims=True), (8, H))
    q = jnp.broadcast_to(jnp.sum(t * t, axis=0, keepdims=True), (8, H))

    @pl.when(k == 0)
    def _():
        s_ref[...] = s
        q_ref[...] = q

    @pl.when(k > 0)
    def _():
        s_ref[...] += s
        q_ref[...] += q


def _stage_a(h, agg2, epsp1, p):
    rb = 2000
    return pl.pallas_call(
        _stage_a_body,
        grid=(N // rb,),
        in_specs=[
            pl.BlockSpec((rb, H), lambda i: (i, 0)),
            pl.BlockSpec((2, rb, H), lambda i: (0, i, 0)),
            pl.BlockSpec((1, H), lambda i: (0, 0)),
            pl.BlockSpec((H, H), lambda i: (0, 0)),
            pl.BlockSpec((1, H), lambda i: (0, 0)),
            pl.BlockSpec((H, H), lambda i: (0, 0)),
            pl.BlockSpec((1, H), lambda i: (0, 0)),
        ],
        out_specs=[
            pl.BlockSpec((rb, H), lambda i: (i, 0)),
            pl.BlockSpec((8, H), lambda i: (0, 0)),
            pl.BlockSpec((8, H), lambda i: (0, 0)),
        ],
        out_shape=[
            jax.ShapeDtypeStruct((N, H), jnp.float32),
            jax.ShapeDtypeStruct((8, H), jnp.float32),
            jax.ShapeDtypeStruct((8, H), jnp.float32),
        ],
    )(h, agg2, epsp1, p["W1"], p["b1"].reshape(1, H), p["W2"],
      p["b2"].reshape(1, H))


def _stage_b_body(t_ref, s_ref, q_ref, g_ref, bt_ref, hres_ref, o_ref):
    mean = s_ref[0:1, :] * (1.0 / N)
    var = q_ref[0:1, :] * (1.0 / N) - mean * mean
    scale = jax.lax.rsqrt(var + 1e-5) * g_ref[...]
    o_ref[...] = (jnp.maximum((t_ref[...] - mean) * scale + bt_ref[...], 0.0)
                  + hres_ref[...])


def _stage_b(t, s, q, gamma, beta, h_res):
    rb = 2000
    return pl.pallas_call(
        _stage_b_body,
        grid=(N // rb,),
        in_specs=[
            pl.BlockSpec((rb, H), lambda i: (i, 0)),
            pl.BlockSpec((8, H), lambda i: (0, 0)),
            pl.BlockSpec((8, H), lambda i: (0, 0)),
            pl.BlockSpec((1, H), lambda i: (0, 0)),
            pl.BlockSpec((1, H), lambda i: (0, 0)),
            pl.BlockSpec((rb, H), lambda i: (i, 0)),
        ],
        out_specs=pl.BlockSpec((rb, H), lambda i: (i, 0)),
        out_shape=jax.ShapeDtypeStruct((N, H), jnp.float32),
    )(t, s, q, gamma.reshape(1, H), beta.reshape(1, H), h_res)


def kernel(x, edge_index, edge_attr, params):
    src = edge_index[0].astype(jnp.int32)
    dst = edge_index[1].astype(jnp.int32)

    h = _proj(x, params["proj_W"], params["proj_b"])
    e = _edge_mlp(edge_attr, params["edge_mlp"])

    for l in range(3):
        p = params["layers"][l]
        agg2 = _sc_agg(h, e, src, dst).reshape(2, N, H)
        epsp1 = jnp.broadcast_to((1.0 + p["eps"]).reshape(1, 1), (1, H))
        t, s, q = _stage_a(h, agg2, epsp1, p["mlp"])
        h = _stage_b(t, s, q, p["gamma"], p["beta"], h)
    return h
